# baseline (speedup 1.0000x reference)
import functools

import jax
import jax.numpy as jnp
from jax import lax
from jax.experimental import pallas as pl
from jax.experimental.pallas import tpu as pltpu
from jax.experimental.pallas import tpu_sc as plsc

_NC = 2
_NS = 16
_NW = _NC * _NS
_L = 16

_V = 24
_NB = 16384
_S = 200
_SB = 8
_S_SC = 8
_S_TC = _S - _S_SC
assert _S_TC % _SB == 0
_BW = _NB // _NW


def _sc_partials(yt1d, pt2d, b_flat):
    mesh = plsc.VectorSubcoreMesh(core_axis_name="c", subcore_axis_name="s")

    @functools.partial(
        pl.kernel,
        out_type=jax.ShapeDtypeStruct((_NW, _L), jnp.float32),
        mesh=mesh,
        scratch_types=[
            pltpu.VMEM((_BW,), jnp.int32),
            pltpu.VMEM((_BW,), jnp.int32),
            pltpu.VMEM((_V, _BW), jnp.float32),
            pltpu.VMEM((_V, _BW), jnp.float32),
            pltpu.VMEM((_V * _V,), jnp.float32),
            pltpu.VMEM((_V * _V,), jnp.float32),
            pltpu.VMEM((_L,), jnp.float32),
            pltpu.SemaphoreType.DMA,
            pltpu.SemaphoreType.DMA,
            pltpu.SemaphoreType.DMA,
            pltpu.SemaphoreType.DMA,
        ],
        compiler_params=pltpu.CompilerParams(needs_layout_passes=False),
    )
    def sc_fn(y_hbm, p_hbm, b_hbm, out_hbm, y_buf0, y_buf1, p_buf0, p_buf1,
              b_vmem, s_vmem, acc_vmem, sem_y0, sem_y1, sem_p0, sem_p1):
        wid = lax.axis_index("s") * _NC + lax.axis_index("c")
        b0 = wid * _BW
        y_bufs = (y_buf0, y_buf1)
        p_bufs = (p_buf0, p_buf1)
        sems_y = (sem_y0, sem_y1)
        sems_p = (sem_p0, sem_p1)

        pltpu.sync_copy(b_hbm, b_vmem)
        col_iota = lax.iota(jnp.int32, _L)

        zero = jnp.zeros((_L,), jnp.float32)
        for v in range(_V * _V // _L):
            s_vmem[pl.ds(v * _L, _L)] = zero

        def _copies(si, buf):
            s = _S_TC + si
            yc = pltpu.make_async_copy(
                y_hbm.at[pl.ds(s * _NB + b0, _BW)], y_bufs[buf], sems_y[buf])
            pc = pltpu.make_async_copy(
                p_hbm.at[pl.ds(s * _V, _V), pl.ds(b0, _BW)], p_bufs[buf],
                sems_p[buf])
            return yc, pc

        def _issue(si, buf):
            yc, pc = _copies(si, buf)
            yc.start()
            pc.start()

        def _compute(si, buf):
            yc, pc = _copies(si, buf)
            yc.wait()
            pc.wait()
            yb = y_bufs[buf]
            pb = p_bufs[buf]

            @plsc.parallel_loop(0, _BW // _L, 1)
            def group_body(g):
                y_v = yb[pl.ds(g * _L, _L)]
                rowoff = y_v * _V
                for k in range(_V):
                    pvec = pb[k, pl.ds(g * _L, _L)]
                    plsc.addupdate_scatter(s_vmem, [rowoff + k], pvec)

        _issue(0, 0)
        _issue(1, 1)

        def s_pair(i, carry):
            c0 = 2 * i
            _compute(c0, 0)

            @pl.when(c0 + 2 < _S_SC)
            def _():
                _issue(c0 + 2, 0)

            _compute(c0 + 1, 1)

            @pl.when(c0 + 3 < _S_SC)
            def _():
                _issue(c0 + 3, 1)

            return carry

        lax.fori_loop(0, _S_SC // 2, s_pair, jnp.int32(0))

        acc0 = zero
        acc1 = zero
        for v in range(_V * _V // _L):
            sv = s_vmem[pl.ds(v * _L, _L)]
            bv = b_vmem[pl.ds(v * _L, _L)]
            if v % 2 == 0:
                acc0 = acc0 + sv * bv
            else:
                acc1 = acc1 + sv * bv
        acc_vmem[...] = acc0 + acc1
        pltpu.sync_copy(acc_vmem, out_hbm.at[wid])

    return sc_fn(yt1d, pt2d, b_flat)


def _tc_sum(yt, pt, B):
    grid = _S_TC // _SB

    def body(y_ref, p_ref, b_ref, out_ref):
        i = pl.program_id(0)
        part = jnp.float32(0.0)
        cls = lax.broadcasted_iota(jnp.int32, (_V, _NB), 0)
        for s in range(_SB):
            ys = y_ref[pl.ds(s, 1), :]
            g = (ys == cls).astype(jnp.float32)
            w = lax.dot_general(
                b_ref[...], g, (((0,), (0,)), ((), ())),
                preferred_element_type=jnp.float32)
            part += jnp.sum(w * p_ref[s])

        @pl.when(i == 0)
        def _():
            out_ref[0, 0] = part

        @pl.when(i > 0)
        def _():
            out_ref[0, 0] += part

    return pl.pallas_call(
        body,
        grid=(grid,),
        in_specs=[
            pl.BlockSpec((_SB, _NB), lambda i: (i, 0)),
            pl.BlockSpec((_SB, _V, _NB), lambda i: (i, 0, 0)),
            pl.BlockSpec((_V, _V), lambda i: (0, 0)),
        ],
        out_specs=pl.BlockSpec(memory_space=pltpu.SMEM),
        out_shape=jax.ShapeDtypeStruct((1, 1), jnp.float32),
        compiler_params=pltpu.CompilerParams(
            dimension_semantics=("arbitrary",)),
    )(yt, pt, B)


def kernel(y_true, y_pred, B):
    yt = y_true.T
    pt = jnp.transpose(y_pred, (1, 2, 0))
    sc_partials = _sc_partials(
        yt.reshape(-1), pt.reshape(_S * _V, _NB), B.reshape(-1))
    tc_part = _tc_sum(yt, pt, B)
    return tc_part[0, 0] + jnp.sum(sc_partials)


# speedup vs baseline: 1.1213x; 1.1213x over previous
import functools

import jax
import jax.numpy as jnp
from jax import lax
from jax.experimental import pallas as pl
from jax.experimental.pallas import tpu as pltpu
from jax.experimental.pallas import tpu_sc as plsc

_NC = 2
_NS = 16
_NW = _NC * _NS
_L = 16

_V = 24
_NB = 16384
_S = 200
_SB = 8
_S_SC = 8
_S_TC = _S - _S_SC
assert _S_TC % _SB == 0
_BW = _NB // _NW


def _sc_partials(yt2d, pt2d, b_flat):
    mesh = plsc.VectorSubcoreMesh(core_axis_name="c", subcore_axis_name="s")

    @functools.partial(
        pl.kernel,
        out_type=jax.ShapeDtypeStruct((_NW, _L), jnp.float32),
        mesh=mesh,
        scratch_types=[
            pltpu.VMEM((_S_SC, _BW), jnp.int32),
            pltpu.VMEM((_V, _BW), jnp.float32),
            pltpu.VMEM((_V, _BW), jnp.float32),
            pltpu.VMEM((_V * _V,), jnp.float32),
            pltpu.VMEM((_V * _V,), jnp.float32),
            pltpu.VMEM((_L,), jnp.float32),
            pltpu.SemaphoreType.DMA,
            pltpu.SemaphoreType.DMA,
            pltpu.SemaphoreType.DMA,
        ],
        compiler_params=pltpu.CompilerParams(needs_layout_passes=False),
    )
    def sc_fn(y_hbm, p_hbm, b_hbm, out_hbm, y_buf, p_buf0, p_buf1,
              b_vmem, s_vmem, acc_vmem, sem_y, sem_p0, sem_p1):
        wid = lax.axis_index("s") * _NC + lax.axis_index("c")
        b0 = wid * _BW
        p_bufs = (p_buf0, p_buf1)
        sems_p = (sem_p0, sem_p1)

        yc = pltpu.make_async_copy(
            y_hbm.at[pl.ds(_S_TC, _S_SC), pl.ds(b0, _BW)], y_buf, sem_y)
        yc.start()

        def _pcopy(si, buf):
            return pltpu.make_async_copy(
                p_hbm.at[pl.ds((_S_TC + si) * _V, _V), pl.ds(b0, _BW)],
                p_bufs[buf], sems_p[buf])

        _pcopy(0, 0).start()
        _pcopy(1, 1).start()

        pltpu.sync_copy(b_hbm, b_vmem)

        zero = jnp.zeros((_L,), jnp.float32)
        for v in range(_V * _V // _L):
            s_vmem[pl.ds(v * _L, _L)] = zero

        yc.wait()

        def _compute(si, buf):
            _pcopy(si, buf).wait()
            pb = p_bufs[buf]

            @plsc.parallel_loop(0, _BW // _L, 1)
            def group_body(g):
                y_v = y_buf[si, pl.ds(g * _L, _L)]
                rowoff = y_v * _V
                for k in range(_V):
                    pvec = pb[k, pl.ds(g * _L, _L)]
                    plsc.addupdate_scatter(s_vmem, [rowoff + k], pvec)

        def s_pair(i, carry):
            c0 = 2 * i
            _compute(c0, 0)

            @pl.when(c0 + 2 < _S_SC)
            def _():
                _pcopy(c0 + 2, 0).start()

            _compute(c0 + 1, 1)

            @pl.when(c0 + 3 < _S_SC)
            def _():
                _pcopy(c0 + 3, 1).start()

            return carry

        lax.fori_loop(0, _S_SC // 2, s_pair, jnp.int32(0))

        acc0 = zero
        acc1 = zero
        for v in range(_V * _V // _L):
            sv = s_vmem[pl.ds(v * _L, _L)]
            bv = b_vmem[pl.ds(v * _L, _L)]
            if v % 2 == 0:
                acc0 = acc0 + sv * bv
            else:
                acc1 = acc1 + sv * bv
        acc_vmem[...] = acc0 + acc1
        pltpu.sync_copy(acc_vmem, out_hbm.at[wid])

    return sc_fn(yt2d, pt2d, b_flat)


def _tc_sum(yt, pt, B):
    grid = _S_TC // _SB

    def body(y_ref, p_ref, b_ref, out_ref):
        i = pl.program_id(0)
        part = jnp.float32(0.0)
        cls = lax.broadcasted_iota(jnp.int32, (_V, _NB), 0)
        for s in range(_SB):
            ys = y_ref[pl.ds(s, 1), :]
            g = (ys == cls).astype(jnp.float32)
            w = lax.dot_general(
                b_ref[...], g, (((0,), (0,)), ((), ())),
                preferred_element_type=jnp.float32)
            part += jnp.sum(w * p_ref[s])

        @pl.when(i == 0)
        def _():
            out_ref[0, 0] = part

        @pl.when(i > 0)
        def _():
            out_ref[0, 0] += part

    return pl.pallas_call(
        body,
        grid=(grid,),
        in_specs=[
            pl.BlockSpec((_SB, _NB), lambda i: (i, 0)),
            pl.BlockSpec((_SB, _V, _NB), lambda i: (i, 0, 0)),
            pl.BlockSpec((_V, _V), lambda i: (0, 0)),
        ],
        out_specs=pl.BlockSpec(memory_space=pltpu.SMEM),
        out_shape=jax.ShapeDtypeStruct((1, 1), jnp.float32),
        compiler_params=pltpu.CompilerParams(
            dimension_semantics=("arbitrary",)),
    )(yt, pt, B)


def kernel(y_true, y_pred, B):
    yt = y_true.T
    pt = jnp.transpose(y_pred, (1, 2, 0))
    sc_partials = _sc_partials(
        yt, pt.reshape(_S * _V, _NB), B.reshape(-1))
    tc_part = _tc_sum(yt, pt, B)
    return tc_part[0, 0] + jnp.sum(sc_partials)
